# baseline (speedup 1.0000x reference)
import functools

import jax
import jax.numpy as jnp
from jax import lax
from jax.experimental import pallas as pl
from jax.experimental.pallas import tpu as pltpu
from jax.experimental.pallas import tpu_sc as plsc

N = 10000
E = 160000
D = 256
H = 128
ALPHA = 0.2
NC, NS, L = 2, 16, 16
NW = NC * NS
CH = 128
NCHUNK = E // CH
CPT = NCHUNK // NS
REM = NCHUNK - CPT * NS
RPT = 624
RCH = 104
RTAIL = N - RPT * NS
RSUM_PAD = 10240

BCH = 8
NBLK = NCHUNK // BCH
BFULL = 28
BREM_TILES = 2

BLK = 1000


def _tc_body(x_ref, w_ref, am_ref, ha_ref, hb_ref, sp_ref):
    h = jnp.dot(x_ref[...], w_ref[...], preferred_element_type=jnp.float32)
    ha_ref[...] = h[:, :H]
    hb_ref[...] = h[:, H:]
    sp_ref[...] = jnp.dot(h, am_ref[...], preferred_element_type=jnp.float32)


_tc_call = pl.pallas_call(
    _tc_body,
    grid=(N // BLK,),
    in_specs=[
        pl.BlockSpec((BLK, D), lambda i: (i, 0)),
        pl.BlockSpec((D, D), lambda i: (0, 0)),
        pl.BlockSpec((D, 2), lambda i: (0, 0)),
    ],
    out_specs=[
        pl.BlockSpec((BLK, H), lambda i: (i, 0)),
        pl.BlockSpec((BLK, H), lambda i: (i, 0)),
        pl.BlockSpec((BLK, 2), lambda i: (i, 0)),
    ],
    out_shape=[
        jax.ShapeDtypeStruct((N, H), jnp.float32),
        jax.ShapeDtypeStruct((N, H), jnp.float32),
        jax.ShapeDtypeStruct((N, 2), jnp.float32),
    ],
)


@functools.partial(
    pl.kernel,
    out_type=[
        jax.ShapeDtypeStruct((N, H), jnp.float32),
        jax.ShapeDtypeStruct((N, H), jnp.float32),
    ],
    mesh=plsc.VectorSubcoreMesh(core_axis_name="c", subcore_axis_name="s"),
    compiler_params=pltpu.CompilerParams(needs_layout_passes=False),
    scratch_types=[
        pltpu.VMEM_SHARED((N, H), jnp.float32),
        pltpu.VMEM_SHARED((RSUM_PAD,), jnp.float32),
        pltpu.VMEM((2, CH), jnp.int32),
        pltpu.VMEM((2, CH), jnp.int32),
        pltpu.VMEM((2, CH), jnp.float32),
        pltpu.VMEM((2, CH), jnp.float32),
        pltpu.VMEM((2, CH), jnp.float32),
        pltpu.VMEM((2, CH, H), jnp.float32),
        pltpu.VMEM((CH,), jnp.float32),
        pltpu.SemaphoreType.DMA,
        pltpu.SemaphoreType.DMA,
        pltpu.SemaphoreType.DMA,
        pltpu.SemaphoreType.DMA,
        pltpu.SemaphoreType.DMA,
        pltpu.SemaphoreType.DMA,
    ],
)
def _sc_agg_kernel(ha, hb, s1, s2, src, dst, outa, outb,
                   acc, rsum, srcv, dstv, s1c, s2c, evc, rows, rsbuf,
                   gsem0, gsem1, ssem0, ssem1, rsem0, rsem1):
    c = lax.axis_index("c")
    s = lax.axis_index("s")
    gsem = (gsem0, gsem1)
    ssem = (ssem0, ssem1)
    rsem = (rsem0, rsem1)

    zv = jnp.zeros((L,), jnp.float32)

    def zero_row(k, carry):
        for q in range(H // L):
            rows[0, k, pl.ds(q * L, L)] = zv
        return carry

    lax.fori_loop(0, CH, zero_row, 0)

    for i in range(6):
        pltpu.sync_copy(rows.at[0, pl.ds(0, RCH)],
                        acc.at[pl.ds(s * RPT + i * RCH, RCH)])

    @pl.when(s == 0)
    def _():
        pltpu.sync_copy(rows.at[0, pl.ds(0, RTAIL)],
                        acc.at[pl.ds(RPT * NS, RTAIL)])

    for i in range(5):
        pltpu.sync_copy(rows.at[0, 0],
                        rsum.at[pl.ds(s * 640 + i * CH, CH)])
    plsc.subcore_barrier()

    base = s * CPT

    def load_idx(g, b):
        eb = (base + g) * CH
        pltpu.sync_copy(src.at[pl.ds(eb, CH)], srcv.at[b])
        pltpu.sync_copy(dst.at[pl.ds(eb, CH)], dstv.at[b])

    def start_gather(g, b):
        pltpu.async_copy(s1.at[srcv.at[b]], s1c.at[b], gsem[b])
        pltpu.async_copy(s2.at[dstv.at[b]], s2c.at[b], gsem[b])

        @pl.when(c == 0)
        def _():
            pltpu.async_copy(ha.at[dstv.at[b]], rows.at[b], gsem[b])

        @pl.when(c == 1)
        def _():
            pltpu.async_copy(hb.at[dstv.at[b]], rows.at[b], gsem[b])

    def wait_gather(g, b):
        pltpu.make_async_copy(s1.at[srcv.at[b]], s1c.at[b], gsem[b]).wait()
        pltpu.make_async_copy(s2.at[dstv.at[b]], s2c.at[b], gsem[b]).wait()
        pltpu.make_async_copy(ha.at[dstv.at[b]], rows.at[b], gsem[b]).wait()

    def start_scatter(b):
        pltpu.async_copy(rows.at[b], acc.at[srcv.at[b]], ssem[b], add=True)

    def wait_scatter(b):
        pltpu.make_async_copy(rows.at[b], acc.at[srcv.at[b]], ssem[b]).wait()

    def start_rowsum(b):
        pltpu.async_copy(evc.at[b], rsum.at[srcv.at[b]], rsem[b], add=True)

    def wait_rowsum(b):
        pltpu.make_async_copy(evc.at[b], rsum.at[srcv.at[b]], rsem[b]).wait()

    def compute_ev(b):
        for j in range(CH // L):
            z = s1c[b, pl.ds(j * L, L)] + s2c[b, pl.ds(j * L, L)]
            zl = jnp.where(z >= 0, z, ALPHA * z)
            evc[b, pl.ds(j * L, L)] = jnp.exp(zl)

    def scale_chunk(b):
        def scale_16(j, carry2):
            e16 = evc[b, pl.ds(j * L, L)]
            for t in range(L):
                ek = e16[t]
                k = j * L + t
                for q in range(H // L):
                    rows[b, k, pl.ds(q * L, L)] = rows[b, k, pl.ds(q * L, L)] * ek
            return carry2

        lax.fori_loop(0, CH // L, scale_16, 0, unroll=2)

    load_idx(0, 0)
    start_gather(0, 0)

    def pair_body(p, carry):
        for b in range(2):
            g2 = 2 * p + b
            b2 = 1 - b

            @pl.when(g2 >= 1)
            def _():
                wait_scatter(b2)
                wait_rowsum(b2)

            @pl.when(g2 < CPT - 1)
            def _():
                load_idx(g2 + 1, b2)
                start_gather(g2 + 1, b2)

            wait_gather(g2, b)
            compute_ev(b)
            start_rowsum(b)
            scale_chunk(b)
            start_scatter(b)
        return carry

    lax.fori_loop(0, CPT // 2, pair_body, 0)
    wait_scatter(1)
    wait_rowsum(1)

    @pl.when(s < REM)
    def _():
        ci = NCHUNK - REM + s
        eb = ci * CH
        pltpu.sync_copy(src.at[pl.ds(eb, CH)], srcv.at[0])
        pltpu.sync_copy(dst.at[pl.ds(eb, CH)], dstv.at[0])
        pltpu.async_copy(s1.at[srcv.at[0]], s1c.at[0], gsem[0])
        pltpu.async_copy(s2.at[dstv.at[0]], s2c.at[0], gsem[0])

        @pl.when(c == 0)
        def _():
            pltpu.async_copy(ha.at[dstv.at[0]], rows.at[0], gsem[0])

        @pl.when(c == 1)
        def _():
            pltpu.async_copy(hb.at[dstv.at[0]], rows.at[0], gsem[0])

        pltpu.make_async_copy(s1.at[srcv.at[0]], s1c.at[0], gsem[0]).wait()
        pltpu.make_async_copy(s2.at[dstv.at[0]], s2c.at[0], gsem[0]).wait()
        pltpu.make_async_copy(ha.at[dstv.at[0]], rows.at[0], gsem[0]).wait()
        compute_ev(0)
        start_rowsum(0)
        scale_chunk(0)
        start_scatter(0)
        wait_scatter(0)
        wait_rowsum(0)

    plsc.subcore_barrier()

    def fin_chunk(r0, nrows):
        pltpu.sync_copy(acc.at[pl.ds(r0, nrows)], rows.at[0, pl.ds(0, nrows)])
        pltpu.sync_copy(rsum.at[pl.ds(r0, CH)], rsbuf)
        for jj in range(CH // L):
            rsbuf[pl.ds(jj * L, L)] = 1.0 / rsbuf[pl.ds(jj * L, L)]

        def fin_row(k, carry):
            inv = plsc.load_gather(rsbuf, [jnp.broadcast_to(k, (L,))])
            for q in range(H // L):
                v = rows[0, k, pl.ds(q * L, L)] * inv
                rows[0, k, pl.ds(q * L, L)] = jnp.where(v > 0, v, jnp.exp(v) - 1.0)
            return carry

        lax.fori_loop(0, nrows, fin_row, 0)

        @pl.when(c == 0)
        def _():
            pltpu.sync_copy(rows.at[0, pl.ds(0, nrows)], outa.at[pl.ds(r0, nrows)])

        @pl.when(c == 1)
        def _():
            pltpu.sync_copy(rows.at[0, pl.ds(0, nrows)], outb.at[pl.ds(r0, nrows)])

    for i in range(6):
        fin_chunk(s * RPT + i * RCH, RCH)

    @pl.when(s == 0)
    def _():
        fin_chunk(RPT * NS, RTAIL)


def kernel(inputs, edge_index, W, a):
    a_mat = a.reshape(2, D).T
    ha, hb, sp = _tc_call(inputs, W, a_mat)
    outa, outb = _sc_agg_kernel(ha, hb, sp[:, 0], sp[:, 1],
                                edge_index[0], edge_index[1])
    return jnp.concatenate([outa, outb], axis=1)


# speedup vs baseline: 1.0359x; 1.0359x over previous
import functools

import jax
import jax.numpy as jnp
from jax import lax
from jax.experimental import pallas as pl
from jax.experimental.pallas import tpu as pltpu
from jax.experimental.pallas import tpu_sc as plsc

N = 10000
E = 160000
D = 256
H = 128
ALPHA = 0.2
NC, NS, L = 2, 16, 16
NW = NC * NS
CH = 128
NCHUNK = E // CH
CPT = NCHUNK // NS
REM = NCHUNK - CPT * NS
RPT = 624
RCH = 104
RTAIL = N - RPT * NS
RSUM_PAD = 10240

BCH = 8
NBLK = NCHUNK // BCH
BFULL = 28
BREM_TILES = 2

BLK = 1000


def _tc_body(x_ref, w_ref, am_ref, ha_ref, hb_ref, sp_ref):
    h = jnp.dot(x_ref[...], w_ref[...], preferred_element_type=jnp.float32)
    ha_ref[...] = h[:, :H]
    hb_ref[...] = h[:, H:]
    sp_ref[...] = jnp.dot(h, am_ref[...], preferred_element_type=jnp.float32)


_tc_call = pl.pallas_call(
    _tc_body,
    grid=(N // BLK,),
    in_specs=[
        pl.BlockSpec((BLK, D), lambda i: (i, 0)),
        pl.BlockSpec((D, D), lambda i: (0, 0)),
        pl.BlockSpec((D, 2), lambda i: (0, 0)),
    ],
    out_specs=[
        pl.BlockSpec((BLK, H), lambda i: (i, 0)),
        pl.BlockSpec((BLK, H), lambda i: (i, 0)),
        pl.BlockSpec((BLK, 2), lambda i: (i, 0)),
    ],
    out_shape=[
        jax.ShapeDtypeStruct((N, H), jnp.float32),
        jax.ShapeDtypeStruct((N, H), jnp.float32),
        jax.ShapeDtypeStruct((N, 2), jnp.float32),
    ],
)


@functools.partial(
    pl.kernel,
    out_type=jax.ShapeDtypeStruct((N, D), jnp.float32),
    mesh=plsc.VectorSubcoreMesh(core_axis_name="c", subcore_axis_name="s"),
    compiler_params=pltpu.CompilerParams(needs_layout_passes=False),
    scratch_types=[
        pltpu.VMEM_SHARED((N, H), jnp.float32),
        pltpu.VMEM_SHARED((RSUM_PAD,), jnp.float32),
        pltpu.VMEM((2, CH), jnp.int32),
        pltpu.VMEM((2, CH), jnp.int32),
        pltpu.VMEM((2, CH), jnp.float32),
        pltpu.VMEM((2, CH), jnp.float32),
        pltpu.VMEM((2, CH), jnp.float32),
        pltpu.VMEM((2, CH, H), jnp.float32),
        pltpu.VMEM((CH,), jnp.float32),
        pltpu.SemaphoreType.DMA,
        pltpu.SemaphoreType.DMA,
        pltpu.SemaphoreType.DMA,
        pltpu.SemaphoreType.DMA,
        pltpu.SemaphoreType.DMA,
        pltpu.SemaphoreType.DMA,
    ],
)
def _sc_agg_kernel(ha, hb, s1, s2, src, dst, out,
                   acc, rsum, srcv, dstv, s1c, s2c, evc, rows, rsbuf,
                   gsem0, gsem1, ssem0, ssem1, rsem0, rsem1):
    c = lax.axis_index("c")
    s = lax.axis_index("s")
    gsem = (gsem0, gsem1)
    ssem = (ssem0, ssem1)
    rsem = (rsem0, rsem1)

    zv = jnp.zeros((L,), jnp.float32)

    def zero_row(k, carry):
        for q in range(H // L):
            rows[0, k, pl.ds(q * L, L)] = zv
        return carry

    lax.fori_loop(0, CH, zero_row, 0)

    for i in range(6):
        pltpu.sync_copy(rows.at[0, pl.ds(0, RCH)],
                        acc.at[pl.ds(s * RPT + i * RCH, RCH)])

    @pl.when(s == 0)
    def _():
        pltpu.sync_copy(rows.at[0, pl.ds(0, RTAIL)],
                        acc.at[pl.ds(RPT * NS, RTAIL)])

    for i in range(5):
        pltpu.sync_copy(rows.at[0, 0],
                        rsum.at[pl.ds(s * 640 + i * CH, CH)])
    plsc.subcore_barrier()

    base = s * CPT

    def load_idx(g, b):
        eb = (base + g) * CH
        pltpu.sync_copy(src.at[pl.ds(eb, CH)], srcv.at[b])
        pltpu.sync_copy(dst.at[pl.ds(eb, CH)], dstv.at[b])

    def start_gather(g, b):
        pltpu.async_copy(s1.at[srcv.at[b]], s1c.at[b], gsem[b])
        pltpu.async_copy(s2.at[dstv.at[b]], s2c.at[b], gsem[b])

        @pl.when(c == 0)
        def _():
            pltpu.async_copy(ha.at[dstv.at[b]], rows.at[b], gsem[b])

        @pl.when(c == 1)
        def _():
            pltpu.async_copy(hb.at[dstv.at[b]], rows.at[b], gsem[b])

    def wait_gather(g, b):
        pltpu.make_async_copy(s1.at[srcv.at[b]], s1c.at[b], gsem[b]).wait()
        pltpu.make_async_copy(s2.at[dstv.at[b]], s2c.at[b], gsem[b]).wait()
        pltpu.make_async_copy(ha.at[dstv.at[b]], rows.at[b], gsem[b]).wait()

    def start_scatter(b):
        pltpu.async_copy(rows.at[b], acc.at[srcv.at[b]], ssem[b], add=True)

    def wait_scatter(b):
        pltpu.make_async_copy(rows.at[b], acc.at[srcv.at[b]], ssem[b]).wait()

    def start_rowsum(b):
        pltpu.async_copy(evc.at[b], rsum.at[srcv.at[b]], rsem[b], add=True)

    def wait_rowsum(b):
        pltpu.make_async_copy(evc.at[b], rsum.at[srcv.at[b]], rsem[b]).wait()

    def compute_ev(b):
        for j in range(CH // L):
            z = s1c[b, pl.ds(j * L, L)] + s2c[b, pl.ds(j * L, L)]
            zl = jnp.where(z >= 0, z, ALPHA * z)
            evc[b, pl.ds(j * L, L)] = jnp.exp(zl)

    def scale_chunk(b):
        def scale_16(j, carry2):
            e16 = evc[b, pl.ds(j * L, L)]
            for t in range(L):
                ek = e16[t]
                k = j * L + t
                for q in range(H // L):
                    rows[b, k, pl.ds(q * L, L)] = rows[b, k, pl.ds(q * L, L)] * ek
            return carry2

        lax.fori_loop(0, CH // L, scale_16, 0)

    load_idx(0, 0)
    start_gather(0, 0)

    def pair_body(p, carry):
        for b in range(2):
            g2 = 2 * p + b
            b2 = 1 - b

            @pl.when(g2 >= 1)
            def _():
                wait_scatter(b2)
                wait_rowsum(b2)

            @pl.when(g2 < CPT - 1)
            def _():
                load_idx(g2 + 1, b2)
                start_gather(g2 + 1, b2)

            wait_gather(g2, b)
            compute_ev(b)
            start_rowsum(b)
            scale_chunk(b)
            start_scatter(b)
        return carry

    lax.fori_loop(0, CPT // 2, pair_body, 0)
    wait_scatter(1)
    wait_rowsum(1)

    @pl.when(s < REM)
    def _():
        ci = NCHUNK - REM + s
        eb = ci * CH
        pltpu.sync_copy(src.at[pl.ds(eb, CH)], srcv.at[0])
        pltpu.sync_copy(dst.at[pl.ds(eb, CH)], dstv.at[0])
        pltpu.async_copy(s1.at[srcv.at[0]], s1c.at[0], gsem[0])
        pltpu.async_copy(s2.at[dstv.at[0]], s2c.at[0], gsem[0])

        @pl.when(c == 0)
        def _():
            pltpu.async_copy(ha.at[dstv.at[0]], rows.at[0], gsem[0])

        @pl.when(c == 1)
        def _():
            pltpu.async_copy(hb.at[dstv.at[0]], rows.at[0], gsem[0])

        pltpu.make_async_copy(s1.at[srcv.at[0]], s1c.at[0], gsem[0]).wait()
        pltpu.make_async_copy(s2.at[dstv.at[0]], s2c.at[0], gsem[0]).wait()
        pltpu.make_async_copy(ha.at[dstv.at[0]], rows.at[0], gsem[0]).wait()
        compute_ev(0)
        start_rowsum(0)
        scale_chunk(0)
        start_scatter(0)
        wait_scatter(0)
        wait_rowsum(0)

    plsc.subcore_barrier()

    def fin_chunk(r0, nrows):
        pltpu.sync_copy(acc.at[pl.ds(r0, nrows)], rows.at[0, pl.ds(0, nrows)])
        pltpu.sync_copy(rsum.at[pl.ds(r0, CH)], rsbuf)
        for jj in range(CH // L):
            rsbuf[pl.ds(jj * L, L)] = 1.0 / rsbuf[pl.ds(jj * L, L)]

        def fin_row(k, carry):
            inv = plsc.load_gather(rsbuf, [jnp.broadcast_to(k, (L,))])
            for q in range(H // L):
                v = rows[0, k, pl.ds(q * L, L)] * inv
                rows[0, k, pl.ds(q * L, L)] = jnp.where(v > 0, v, jnp.exp(v) - 1.0)
            return carry

        lax.fori_loop(0, nrows, fin_row, 0)

        @pl.when(c == 0)
        def _():
            pltpu.sync_copy(rows.at[0, pl.ds(0, nrows)],
                            out.at[pl.ds(r0, nrows), pl.ds(0, H)])

        @pl.when(c == 1)
        def _():
            pltpu.sync_copy(rows.at[0, pl.ds(0, nrows)],
                            out.at[pl.ds(r0, nrows), pl.ds(H, H)])

    for i in range(6):
        fin_chunk(s * RPT + i * RCH, RCH)

    @pl.when(s == 0)
    def _():
        fin_chunk(RPT * NS, RTAIL)


def kernel(inputs, edge_index, W, a):
    a_mat = a.reshape(2, D).T
    ha, hb, sp = _tc_call(inputs, W, a_mat)
    return _sc_agg_kernel(ha, hb, sp[:, 0], sp[:, 1],
                          edge_index[0], edge_index[1])
